# baseline (speedup 1.0000x reference)
import functools

import jax
import jax.numpy as jnp
from jax.experimental import pallas as pl
from jax.experimental.pallas import tpu as pltpu

NUM_EXPERTS = 8
TOP_K = 2


def _router_body(x_ref, gw_ref, logits_ref, gates_ref):
    x = x_ref[...]
    gw = gw_ref[...]
    logits = jax.lax.dot_general(
        x, gw, (((1,), (1,)), ((), ())), preferred_element_type=jnp.float32
    )
    logits_ref[...] = logits
    m = jnp.max(logits, axis=1, keepdims=True)
    p = jnp.exp(logits - m)
    p = p / jnp.sum(p, axis=1, keepdims=True)
    ei = jax.lax.broadcasted_iota(jnp.int32, p.shape, 1)
    a1 = jnp.argmax(p, axis=1)[:, None]
    mask1 = ei == a1
    p2 = jnp.where(mask1, -1.0, p)
    a2 = jnp.argmax(p2, axis=1)[:, None]
    mask2 = ei == a2
    v1 = jnp.sum(jnp.where(mask1, p, 0.0), axis=1, keepdims=True)
    v2 = jnp.sum(jnp.where(mask2, p, 0.0), axis=1, keepdims=True)
    denom = v1 + v2
    gates_ref[...] = jnp.where(mask1 | mask2, p, 0.0) / denom


def _expert_body(xb_ref, w1_ref, w2_ref, w3_ref, g_ref, out_ref, acc_ref, *, bt, ne):
    e = pl.program_id(0)
    t = pl.program_id(1)
    x = xb_ref[...]
    w1e = w1_ref[0]
    w3e = w3_ref[0]
    w2e = w2_ref[0]
    h1 = jax.lax.dot_general(
        x, w1e, (((1,), (1,)), ((), ())), preferred_element_type=jnp.float32
    )
    h3 = jax.lax.dot_general(
        x, w3e, (((1,), (1,)), ((), ())), preferred_element_type=jnp.float32
    )
    h = (h1 * jax.nn.sigmoid(h1)) * h3
    hb = h.astype(jnp.bfloat16)
    y = jax.lax.dot_general(
        hb, w2e, (((1,), (1,)), ((), ())), preferred_element_type=jnp.float32
    )
    ei = jax.lax.broadcasted_iota(jnp.int32, g_ref.shape, 1)
    g = jnp.sum(jnp.where(ei == e, g_ref[...], 0.0), axis=1)
    contrib = y * g[:, None]
    sl = pl.ds(t * bt, bt)

    @pl.when(e == 0)
    def _():
        acc_ref[sl, :] = contrib

    @pl.when(e != 0)
    def _():
        acc_ref[sl, :] = acc_ref[sl, :] + contrib

    @pl.when(e == ne - 1)
    def _():
        out_ref[...] = acc_ref[sl, :]


def kernel(hidden_states, gate_w, w1, w2, w3):
    B, S, D = hidden_states.shape
    T = B * S
    E, F, _ = w1.shape
    x = hidden_states.reshape(T, D)

    logits, gates = pl.pallas_call(
        _router_body,
        out_shape=[
            jax.ShapeDtypeStruct((T, E), jnp.float32),
            jax.ShapeDtypeStruct((T, E), jnp.float32),
        ],
    )(x, gate_w)

    xb = x.astype(jnp.bfloat16)
    w1b = w1.astype(jnp.bfloat16)
    w2b = w2.astype(jnp.bfloat16)
    w3b = w3.astype(jnp.bfloat16)

    BT = 256
    TB = T // BT
    body = functools.partial(_expert_body, bt=BT, ne=E)
    out = pl.pallas_call(
        body,
        grid=(E, TB),
        in_specs=[
            pl.BlockSpec((BT, D), lambda e, t: (t, 0)),
            pl.BlockSpec((1, F, D), lambda e, t: (e, 0, 0)),
            pl.BlockSpec((1, D, F), lambda e, t: (e, 0, 0)),
            pl.BlockSpec((1, F, D), lambda e, t: (e, 0, 0)),
            pl.BlockSpec((BT, E), lambda e, t: (t, 0)),
        ],
        out_specs=pl.BlockSpec((BT, D), lambda e, t: (t, 0)),
        out_shape=jax.ShapeDtypeStruct((T, D), jnp.float32),
        scratch_shapes=[pltpu.VMEM((T, D), jnp.float32)],
    )(xb, w1b, w2b, w3b, gates)

    return out.reshape(B, S, D), logits


# speedup vs baseline: 1.5568x; 1.5568x over previous
import functools

import jax
import jax.numpy as jnp
from jax import lax
from jax.experimental import pallas as pl
from jax.experimental.pallas import tpu as pltpu
from jax.experimental.pallas import tpu_sc as plsc

NUM_EXPERTS = 8
TOP_K = 2
BT = 256
BSH = BT.bit_length() - 1


def _router_body(x_ref, gw_ref, logits_ref, sel_ref, wts_ref):
    x = x_ref[...]
    gw = gw_ref[...]
    logits = jax.lax.dot_general(
        x, gw, (((1,), (1,)), ((), ())), preferred_element_type=jnp.float32
    )
    logits_ref[...] = logits
    m = jnp.max(logits, axis=1, keepdims=True)
    p = jnp.exp(logits - m)
    p = p / jnp.sum(p, axis=1, keepdims=True)
    ei = jax.lax.broadcasted_iota(jnp.int32, p.shape, 1)
    a1 = jnp.argmax(p, axis=1)[:, None]
    mask1 = ei == a1
    p2 = jnp.where(mask1, -1.0, p)
    a2 = jnp.argmax(p2, axis=1)[:, None]
    mask2 = ei == a2
    v1 = jnp.sum(jnp.where(mask1, p, 0.0), axis=1, keepdims=True)
    v2 = jnp.sum(jnp.where(mask2, p, 0.0), axis=1, keepdims=True)
    denom = v1 + v2
    sel_ref[...] = jnp.concatenate([a1, a2], axis=1).astype(jnp.int32)
    wts_ref[...] = jnp.concatenate([v1, v2], axis=1) / denom


def _bucket_body(np_, nblk, npad, eids_hbm, stok_hbm, smap_hbm, be_hbm,
                 nbu_hbm, eids_v, stok_v, smap_v, be_v, nbu_v, offref):
    wid = lax.axis_index("s") * 2 + lax.axis_index("c")

    @pl.when(wid == 0)
    def _():
        pltpu.sync_copy(eids_hbm, eids_v)
        li = lax.broadcasted_iota(jnp.int32, (16,), 0)
        zeros = jnp.zeros((16,), jnp.int32)

        def zbody(i, _):
            stok_v[pl.ds(i * 16, 16)] = zeros
            return 0

        lax.fori_loop(0, (npad + 16) // 16, zbody, 0)

        def cbody(c, cnt):
            ev = eids_v[pl.ds(c * 16, 16)]
            for e in range(NUM_EXPERTS):
                pc = jnp.sum((ev == e).astype(jnp.int32))
                cnt = cnt + jnp.where(li == e, pc, 0)
            return cnt

        cnt = lax.fori_loop(0, np_ // 16, cbody, zeros)
        pcv = ((cnt + (BT - 1)) >> BSH) << BSH
        ic = plsc.cumsum(pcv)
        start = ic - pcv
        for e in range(NUM_EXPERTS):
            offref[e] = start[e]

        bu = ic[NUM_EXPERTS - 1] >> BSH
        lastbe = jnp.max(jnp.where(cnt > 0, li, 0))
        for cc in range((nblk + 15) // 16):
            b = li + cc * 16
            bev = jnp.zeros((16,), jnp.int32)
            for e in range(NUM_EXPERTS):
                sb = start[e] >> BSH
                ub = pcv[e] >> BSH
                bev = jnp.where((b >= sb) & (b < sb + ub), e, bev)
            bev = jnp.where(b < bu, bev, lastbe)
            be_v[pl.ds(cc * 16, 16)] = bev

        def pbody(c, _):
            ev = eids_v[pl.ds(c * 16, 16)]
            iv = li + c * 16
            tv = iv >> 1
            posacc = jnp.zeros((16,), jnp.int32)
            for e in range(NUM_EXPERTS):
                mask = ev == e
                m32 = mask.astype(jnp.int32)
                off = offref[e]
                plsc.store_compressed(stok_v.at[pl.ds(off, 16)], tv, mask=mask)
                pos = off + plsc.cumsum(m32) - 1
                posacc = jnp.where(mask, pos, posacc)
                offref[e] = off + jnp.sum(m32)
            smap_v[pl.ds(c * 16, 16)] = posacc
            return 0

        lax.fori_loop(0, np_ // 16, pbody, 0)

        nbu_v[...] = jnp.where(li >= 0, bu, 0)
        pltpu.sync_copy(stok_v.at[pl.ds(0, npad)], stok_hbm)
        pltpu.sync_copy(smap_v, smap_hbm)
        pltpu.sync_copy(be_v, be_hbm)
        pltpu.sync_copy(nbu_v, nbu_hbm)


def _gather_body(npad, t, dh, x_hbm, stok_hbm, xs_hbm, idx_v, rows_v, shared,
                 gsem0, gsem1, wsem0, wsem1):
    cid = lax.axis_index("c")
    sid = lax.axis_index("s")
    wid = sid * 2 + cid
    per_w = npad // 32
    base = wid * per_w
    stripe = t // 16
    pltpu.sync_copy(x_hbm.at[pl.ds(sid * stripe, stripe), :],
                    shared.at[pl.ds(sid * stripe, stripe), :])
    pltpu.sync_copy(stok_hbm.at[pl.ds(base, per_w)], idx_v)
    plsc.subcore_barrier()
    nfull, rem = divmod(per_w, 24)
    sizes = [24] * nfull + ([rem] if rem else [])
    gsems = (gsem0, gsem1, wsem1)
    gi = 0
    goff = 0
    while gi < len(sizes):
        grp = sizes[gi:gi + 4]
        gh = []
        coff = 0
        for k, sz in enumerate(grp):
            gh.append(pltpu.async_copy(
                shared.at[idx_v.at[pl.ds(goff + coff, sz)]],
                rows_v.at[pl.ds(coff, sz), :],
                gsems[k % 3]))
            coff += sz
        for g in gh:
            g.wait()
        pltpu.async_copy(
            rows_v.at[pl.ds(0, coff), :],
            xs_hbm.at[pl.ds(base + goff, coff), :], wsem0).wait()
        goff += coff
        gi += 4


def _expert_body(nblk, bt, be_ref, stok_ref, nbu_ref, xi_ref, w1_ref, w2_ref,
                 w3_ref, out_ref, xbuf, sem0, sem1):
    b = pl.program_id(0)
    sems = (sem0, sem1)
    used = nbu_ref[0]

    def issue(blk, bufi):
        def row_body(i, _):
            for k in range(4):
                r = i * 4 + k
                row = stok_ref[blk * bt + r]
                pltpu.make_async_copy(
                    xi_ref.at[pl.ds(row, 1), :],
                    xbuf.at[bufi, pl.ds(r, 1), :],
                    sems[bufi]).start()
            return 0

        lax.fori_loop(0, bt // 4, row_body, 0)

    def drain(bufi):
        pltpu.make_async_copy(
            xi_ref.at[pl.ds(0, bt), :], xbuf.at[bufi], sems[bufi]).wait()

    @pl.when(b == 0)
    def _():
        issue(0, 0)

    nxt = b + 1

    @pl.when((nxt < used) & (nxt % 2 == 0))
    def _():
        issue(nxt, 0)

    @pl.when((nxt < used) & (nxt % 2 == 1))
    def _():
        issue(nxt, 1)

    cur = b % 2

    @pl.when((b < used) & (cur == 0))
    def _():
        drain(0)

    @pl.when((b < used) & (cur == 1))
    def _():
        drain(1)

    @pl.when(b < used)
    def _():
        x = xbuf[cur].astype(jnp.bfloat16)
        w1e = w1_ref[0]
        w3e = w3_ref[0]
        w2e = w2_ref[0]
        h1 = jax.lax.dot_general(
            x, w1e, (((1,), (1,)), ((), ())),
            preferred_element_type=jnp.float32)
        h3 = jax.lax.dot_general(
            x, w3e, (((1,), (1,)), ((), ())),
            preferred_element_type=jnp.float32)
        h = (h1 * jax.nn.sigmoid(h1)) * h3
        hb = h.astype(jnp.bfloat16)
        out_ref[...] = jax.lax.dot_general(
            hb, w2e, (((1,), (1,)), ((), ())),
            preferred_element_type=jnp.float32)


def _combine_body(t, d, y_hbm, smap_hbm, wflat_hbm, out_hbm,
                  sidx_v, wv_v, rows_v, ov_v, sem):
    wid = lax.axis_index("s") * 2 + lax.axis_index("c")
    per_w = t // 32
    base_t = wid * per_w
    pltpu.sync_copy(smap_hbm.at[pl.ds(2 * base_t, 2 * per_w)], sidx_v)
    pltpu.sync_copy(wflat_hbm.at[pl.ds(2 * base_t, 2 * per_w)], wv_v)
    for j in range(per_w // 16):
        pltpu.async_copy(
            y_hbm.at[sidx_v.at[pl.ds(32 * j, 32)]], rows_v, sem).wait()
        wa = wv_v[pl.ds(32 * j, 16)]
        wb = wv_v[pl.ds(32 * j + 16, 16)]
        ws = [wa[i] for i in range(16)] + [wb[i] for i in range(16)]

        def cbody(cc, _):
            sl = pl.ds(cc * 16, 16)
            for tt in range(16):
                o = (ws[2 * tt] * rows_v[2 * tt, sl]
                     + ws[2 * tt + 1] * rows_v[2 * tt + 1, sl])
                ov_v[tt, sl] = o
            return 0

        lax.fori_loop(0, d // 16, cbody, 0)
        pltpu.sync_copy(ov_v, out_hbm.at[pl.ds(base_t + 16 * j, 16), :])


def kernel(hidden_states, gate_w, w1, w2, w3):
    B, S, D = hidden_states.shape
    T = B * S
    E, F, _ = w1.shape
    NP = T * TOP_K
    NBLK = NP // BT + (NUM_EXPERTS - 1)
    NBLK += (-NBLK) % (256 // BT)
    NPAD = NBLK * BT
    NBE = ((NBLK + 15) // 16) * 16
    x = hidden_states.reshape(T, D)

    logits, sel, wts = pl.pallas_call(
        _router_body,
        out_shape=[
            jax.ShapeDtypeStruct((T, E), jnp.float32),
            jax.ShapeDtypeStruct((T, TOP_K), jnp.int32),
            jax.ShapeDtypeStruct((T, TOP_K), jnp.float32),
        ],
    )(x, gate_w)
    eids = sel.reshape(NP)
    wflat = wts.reshape(NP)

    mesh = plsc.VectorSubcoreMesh(core_axis_name="c", subcore_axis_name="s")

    bucket = functools.partial(
        pl.kernel,
        mesh=mesh,
        compiler_params=pltpu.CompilerParams(needs_layout_passes=False),
        out_type=[
            jax.ShapeDtypeStruct((NPAD,), jnp.int32),
            jax.ShapeDtypeStruct((NP,), jnp.int32),
            jax.ShapeDtypeStruct((NBE,), jnp.int32),
            jax.ShapeDtypeStruct((16,), jnp.int32),
        ],
        scratch_types=[
            pltpu.VMEM((NP,), jnp.int32),
            pltpu.VMEM((NPAD + 16,), jnp.int32),
            pltpu.VMEM((NP,), jnp.int32),
            pltpu.VMEM((NBE,), jnp.int32),
            pltpu.VMEM((16,), jnp.int32),
            pltpu.SMEM((NUM_EXPERTS,), jnp.int32),
        ],
    )(functools.partial(_bucket_body, NP, NBLK, NPAD))
    stok, smap, be, nbu = bucket(eids)

    w1b = w1.astype(jnp.bfloat16)
    w2b = w2.astype(jnp.bfloat16)
    w3b = w3.astype(jnp.bfloat16)

    y = pl.pallas_call(
        functools.partial(_expert_body, NBLK, BT),
        grid_spec=pltpu.PrefetchScalarGridSpec(
            num_scalar_prefetch=3,
            grid=(NBLK,),
            in_specs=[
                pl.BlockSpec(memory_space=pl.ANY),
                pl.BlockSpec((1, F, D),
                             lambda b, be_ref, st_ref, nb_ref: (be_ref[b], 0, 0)),
                pl.BlockSpec((1, D, F),
                             lambda b, be_ref, st_ref, nb_ref: (be_ref[b], 0, 0)),
                pl.BlockSpec((1, F, D),
                             lambda b, be_ref, st_ref, nb_ref: (be_ref[b], 0, 0)),
            ],
            out_specs=pl.BlockSpec((BT, D),
                                   lambda b, be_ref, st_ref, nb_ref: (b, 0)),
            scratch_shapes=[
                pltpu.VMEM((2, BT, D), jnp.float32),
                pltpu.SemaphoreType.DMA,
                pltpu.SemaphoreType.DMA,
            ],
        ),
        out_shape=jax.ShapeDtypeStruct((NPAD, D), jnp.float32),
    )(be, stok, nbu, x, w1b, w2b, w3b)

    combine = functools.partial(
        pl.kernel,
        mesh=mesh,
        compiler_params=pltpu.CompilerParams(needs_layout_passes=False),
        out_type=jax.ShapeDtypeStruct((T, D), jnp.float32),
        scratch_types=[
            pltpu.VMEM((2 * (T // 32),), jnp.int32),
            pltpu.VMEM((2 * (T // 32),), jnp.float32),
            pltpu.VMEM((32, D), jnp.float32),
            pltpu.VMEM((16, D), jnp.float32),
            pltpu.SemaphoreType.DMA,
        ],
    )(functools.partial(_combine_body, T, D))
    final = combine(y, smap, wflat)

    return final.reshape(B, S, D), logits
